# baseline (speedup 1.0000x reference)
import functools

import jax
import jax.numpy as jnp
from jax import lax
from jax.experimental import pallas as pl
from jax.experimental.pallas import tpu as pltpu
from jax.experimental.pallas import tpu_sc as plsc

D = 128
B = 16384

_info = plsc.get_sparse_core_info()
NC, NS, L = _info.num_cores, _info.num_subcores, _info.num_lanes
NW = NC * NS
E_PER_W = B // NW
CHUNK = 128
N_CHUNKS = E_PER_W // CHUNK

_IB = lax.GatherScatterMode.PROMISE_IN_BOUNDS


_DNUMS = lax.GatherDimensionNumbers(
    offset_dims=(), collapsed_slice_dims=(0,), start_index_map=(0,))


def _perm(x, idx):
    return lax.gather(x, idx[:, None], _DNUMS, slice_sizes=(1,), mode=_IB)


def _sums_kernel(heads_hbm, tails_hbm, in_hbm, out_hbm, w_hbm, sums_hbm,
                 hv, tv, ub, vb, wb, ob, sem_u, sem_v):
    wid = lax.axis_index("s") * NC + lax.axis_index("c")
    lane = lax.iota(jnp.int32, L)
    lane0 = lane == 0
    hi_idx = (lane & 7) + 8
    p4_idx = (lane & 3) + 4
    p2_idx = (lane & 1) + 2
    p1_idx = (lane & 0) + 1

    pltpu.sync_copy(w_hbm, wb)
    wv = [wb[pl.ds(16 * j, 16)] for j in range(8)]

    def chunk_body(ci, carry):
        base = wid * E_PER_W + ci * CHUNK
        pltpu.sync_copy(heads_hbm.at[pl.ds(base, CHUNK)], hv)
        pltpu.sync_copy(tails_hbm.at[pl.ds(base, CHUNK)], tv)
        cp_u = pltpu.async_copy(in_hbm.at[hv], ub, sem_u)
        cp_v = pltpu.async_copy(out_hbm.at[tv], vb, sem_v)
        cp_u.wait()
        cp_v.wait()

        def edge_body(e, c2):
            acc = None
            for j in range(8):
                uu = ub[e, pl.ds(16 * j, 16)]
                vv = vb[e, pl.ds(16 * j, 16)]
                m = (uu * vv) * wv[j]
                acc = m if j == 0 else acc + m
                acc = acc + _perm(m, hi_idx)
            mm = acc + _perm(acc, p4_idx)
            nn = mm + _perm(mm, p2_idx)
            res = nn + _perm(nn, p1_idx)
            plsc.store_scatter(ob, [jnp.zeros((L,), jnp.int32) + e], res,
                               mask=lane0)
            return c2

        lax.fori_loop(0, CHUNK, edge_body, 0, unroll=False)
        pltpu.sync_copy(ob, sums_hbm.at[pl.ds(base, CHUNK)])
        return carry

    lax.fori_loop(0, N_CHUNKS, chunk_body, 0, unroll=False)


@jax.jit
def _edge_sums(heads, tails, in_embed, out_embed, diag_w):
    mesh = plsc.VectorSubcoreMesh(core_axis_name="c", subcore_axis_name="s")
    k = functools.partial(
        pl.kernel,
        mesh=mesh,
        compiler_params=pltpu.CompilerParams(needs_layout_passes=False),
        out_type=jax.ShapeDtypeStruct((B,), jnp.float32),
        scratch_types=[
            pltpu.VMEM((CHUNK,), jnp.int32),
            pltpu.VMEM((CHUNK,), jnp.int32),
            pltpu.VMEM((CHUNK, D), jnp.float32),
            pltpu.VMEM((CHUNK, D), jnp.float32),
            pltpu.VMEM((D,), jnp.float32),
            pltpu.VMEM((CHUNK,), jnp.float32),
            pltpu.SemaphoreType.DMA,
            pltpu.SemaphoreType.DMA,
        ],
    )(_sums_kernel)
    return k(heads, tails, in_embed, out_embed, diag_w)


def kernel(heads, tails, in_embed, out_embed, diag_w):
    sums = _edge_sums(heads, tails, in_embed, out_embed, diag_w)
    log_target = jax.nn.sigmoid(sums)
    order = jnp.argsort(-log_target)
    return log_target, order
